# baseline (speedup 1.0000x reference)
import functools

import jax
import jax.numpy as jnp
from jax.experimental import pallas as pl
from jax.experimental.pallas import tpu as pltpu
from jax.experimental.pallas import tpu_sc as plsc

_SHAPE = (2, 16384, 1024)
_FLAT_ROWS = 2 * _SHAPE[1]
_N_CHUNKS = 16
_CHUNK_ROWS = _FLAT_ROWS // _N_CHUNKS
_NBUF = 4
_PATCH_CHUNK = _SHAPE[1] // _CHUNK_ROWS
_TILE = (8, 128)

_sc_mesh = plsc.VectorSubcoreMesh(core_axis_name="c", subcore_axis_name="s")


@functools.partial(
    pl.kernel,
    mesh=_sc_mesh,
    out_type=jax.ShapeDtypeStruct(_TILE, jnp.float32),
    scratch_types=[
        pltpu.VMEM(_TILE, jnp.float32),
        pltpu.VMEM((16,), jnp.float32),
    ],
)
def _sc_rmw(key_hbm, cache_hbm, patch_hbm, tile_v, key_v):
    cid = jax.lax.axis_index("c")
    sid = jax.lax.axis_index("s")

    @pl.when((cid == 0) & (sid == 0))
    def _():
        pltpu.sync_copy(key_hbm, key_v)
        pltpu.sync_copy(
            cache_hbm.at[1, pl.ds(0, _TILE[0]), pl.ds(0, _TILE[1])], tile_v
        )
        lane = jax.lax.iota(jnp.int32, 16)
        chunk = tile_v[0, pl.ds(0, 16)]
        tile_v[0, pl.ds(0, 16)] = chunk + jnp.where(
            lane == 1, 2.0 * key_v[...], 0.0
        )
        pltpu.sync_copy(tile_v, patch_hbm)


def _copy_stitch_kernel(in_ref, patch_ref, out_ref, bufs, sem_in, sem_out):
    def start_in(i):
        pltpu.make_async_copy(
            in_ref.at[pl.ds(i * _CHUNK_ROWS, _CHUNK_ROWS), :],
            bufs.at[i % _NBUF],
            sem_in.at[i % _NBUF],
        ).start()

    def wait_in(i):
        pltpu.make_async_copy(
            in_ref.at[pl.ds(i * _CHUNK_ROWS, _CHUNK_ROWS), :],
            bufs.at[i % _NBUF],
            sem_in.at[i % _NBUF],
        ).wait()

    def start_out(i):
        pltpu.make_async_copy(
            bufs.at[i % _NBUF],
            out_ref.at[pl.ds(i * _CHUNK_ROWS, _CHUNK_ROWS), :],
            sem_out.at[i % _NBUF],
        ).start()

    def wait_out(i):
        pltpu.make_async_copy(
            bufs.at[i % _NBUF],
            out_ref.at[pl.ds(i * _CHUNK_ROWS, _CHUNK_ROWS), :],
            sem_out.at[i % _NBUF],
        ).wait()

    lookahead = _NBUF // 2
    for i in range(lookahead):
        start_in(i)
    for i in range(_N_CHUNKS):
        nxt = i + lookahead
        if nxt < _N_CHUNKS:
            if nxt >= _NBUF:
                wait_out(nxt - _NBUF)
            start_in(nxt)
        wait_in(i)
        if i == _PATCH_CHUNK:
            bufs[i % _NBUF, 0 : _TILE[0], 0 : _TILE[1]] = patch_ref[...]
        start_out(i)
    for i in range(_N_CHUNKS - _NBUF, _N_CHUNKS):
        wait_out(i)


def kernel(key, cache_next):
    key16 = jnp.broadcast_to(key, (16,))
    patch = _sc_rmw(key16, cache_next)
    flat = cache_next.reshape(_FLAT_ROWS, _SHAPE[2])
    out = pl.pallas_call(
        _copy_stitch_kernel,
        out_shape=jax.ShapeDtypeStruct((_FLAT_ROWS, _SHAPE[2]), jnp.float32),
        in_specs=[
            pl.BlockSpec(memory_space=pl.ANY),
            pl.BlockSpec(memory_space=pltpu.VMEM),
        ],
        out_specs=pl.BlockSpec(memory_space=pl.ANY),
        scratch_shapes=[
            pltpu.VMEM((_NBUF, _CHUNK_ROWS, _SHAPE[2]), jnp.float32),
            pltpu.SemaphoreType.DMA((_NBUF,)),
            pltpu.SemaphoreType.DMA((_NBUF,)),
        ],
    )(flat, patch)
    return key, out.reshape(_SHAPE)


# speedup vs baseline: 1.2403x; 1.2403x over previous
import jax
import jax.numpy as jnp
from jax.experimental import pallas as pl
from jax.experimental.pallas import tpu as pltpu

_SHAPE = (2, 16384, 1024)
_FLAT_ROWS = 2 * _SHAPE[1]
_N_CHUNKS = 16
_CHUNK_ROWS = _FLAT_ROWS // _N_CHUNKS
_NBUF = 6
_PATCH_CHUNK = _SHAPE[1] // _CHUNK_ROWS


def _copy_update_kernel(key_ref, in_ref, out_ref, bufs, sem_in, sem_out):
    def start_in(i):
        pltpu.make_async_copy(
            in_ref.at[pl.ds(i * _CHUNK_ROWS, _CHUNK_ROWS), :],
            bufs.at[i % _NBUF],
            sem_in.at[i % _NBUF],
        ).start()

    def wait_in(i):
        pltpu.make_async_copy(
            in_ref.at[pl.ds(i * _CHUNK_ROWS, _CHUNK_ROWS), :],
            bufs.at[i % _NBUF],
            sem_in.at[i % _NBUF],
        ).wait()

    def start_out(i):
        pltpu.make_async_copy(
            bufs.at[i % _NBUF],
            out_ref.at[pl.ds(i * _CHUNK_ROWS, _CHUNK_ROWS), :],
            sem_out.at[i % _NBUF],
        ).start()

    def wait_out(i):
        pltpu.make_async_copy(
            bufs.at[i % _NBUF],
            out_ref.at[pl.ds(i * _CHUNK_ROWS, _CHUNK_ROWS), :],
            sem_out.at[i % _NBUF],
        ).wait()

    lookahead = _NBUF // 2
    for i in range(lookahead):
        start_in(i)
    for i in range(_N_CHUNKS):
        nxt = i + lookahead
        if nxt < _N_CHUNKS:
            if nxt >= _NBUF:
                wait_out(nxt - _NBUF)
            start_in(nxt)
        wait_in(i)
        if i == _PATCH_CHUNK:
            row = jax.lax.broadcasted_iota(jnp.int32, (8, 128), 0)
            col = jax.lax.broadcasted_iota(jnp.int32, (8, 128), 1)
            mask = (row == 0) & (col == 1)
            bufs[i % _NBUF, 0:8, 0:128] += jnp.where(
                mask, 2.0 * key_ref[0], 0.0
            )
        start_out(i)
    for i in range(_N_CHUNKS - _NBUF, _N_CHUNKS):
        wait_out(i)


def kernel(key, cache_next):
    flat = cache_next.reshape(_FLAT_ROWS, _SHAPE[2])
    out = pl.pallas_call(
        _copy_update_kernel,
        out_shape=jax.ShapeDtypeStruct((_FLAT_ROWS, _SHAPE[2]), jnp.float32),
        in_specs=[
            pl.BlockSpec(memory_space=pltpu.SMEM),
            pl.BlockSpec(memory_space=pl.ANY),
        ],
        out_specs=pl.BlockSpec(memory_space=pl.ANY),
        scratch_shapes=[
            pltpu.VMEM((_NBUF, _CHUNK_ROWS, _SHAPE[2]), jnp.float32),
            pltpu.SemaphoreType.DMA((_NBUF,)),
            pltpu.SemaphoreType.DMA((_NBUF,)),
        ],
    )(key, flat)
    return key, out.reshape(_SHAPE)


# speedup vs baseline: 1.2484x; 1.0066x over previous
import jax
import jax.numpy as jnp
from jax.experimental import pallas as pl
from jax.experimental.pallas import tpu as pltpu

_SHAPE = (2, 16384, 1024)
_FLAT_ROWS = 2 * _SHAPE[1]
_CHUNK_ROWS = [512, 1024, 2048] + [3200] * 8 + [2048, 1024, 512]
_OFFSETS = [sum(_CHUNK_ROWS[:i]) for i in range(len(_CHUNK_ROWS))]
assert sum(_CHUNK_ROWS) == _FLAT_ROWS
_N_CHUNKS = len(_CHUNK_ROWS)
_MAX_ROWS = max(_CHUNK_ROWS)
_NBUF = 4
_PATCH_ROW = _SHAPE[1]
_PATCH_CHUNK = next(
    i
    for i, (o, r) in enumerate(zip(_OFFSETS, _CHUNK_ROWS))
    if o <= _PATCH_ROW and _PATCH_ROW + 8 <= o + r
)


def _copy_update_kernel(key_ref, in_ref, out_ref, bufs, sem_in, sem_out):
    def start_in(i):
        pltpu.make_async_copy(
            in_ref.at[pl.ds(_OFFSETS[i], _CHUNK_ROWS[i]), :],
            bufs.at[i % _NBUF, pl.ds(0, _CHUNK_ROWS[i]), :],
            sem_in.at[i % _NBUF],
        ).start()

    def wait_in(i):
        pltpu.make_async_copy(
            in_ref.at[pl.ds(_OFFSETS[i], _CHUNK_ROWS[i]), :],
            bufs.at[i % _NBUF, pl.ds(0, _CHUNK_ROWS[i]), :],
            sem_in.at[i % _NBUF],
        ).wait()

    def start_out(i):
        pltpu.make_async_copy(
            bufs.at[i % _NBUF, pl.ds(0, _CHUNK_ROWS[i]), :],
            out_ref.at[pl.ds(_OFFSETS[i], _CHUNK_ROWS[i]), :],
            sem_out.at[i % _NBUF],
        ).start()

    def wait_out(i):
        pltpu.make_async_copy(
            bufs.at[i % _NBUF, pl.ds(0, _CHUNK_ROWS[i]), :],
            out_ref.at[pl.ds(_OFFSETS[i], _CHUNK_ROWS[i]), :],
            sem_out.at[i % _NBUF],
        ).wait()

    lookahead = _NBUF // 2
    for i in range(lookahead):
        start_in(i)
    for i in range(_N_CHUNKS):
        nxt = i + lookahead
        if nxt < _N_CHUNKS:
            if nxt >= _NBUF:
                wait_out(nxt - _NBUF)
            start_in(nxt)
        wait_in(i)
        if i == _PATCH_CHUNK:
            base = _PATCH_ROW - _OFFSETS[i]
            row = jax.lax.broadcasted_iota(jnp.int32, (8, 128), 0)
            col = jax.lax.broadcasted_iota(jnp.int32, (8, 128), 1)
            mask = (row == 0) & (col == 1)
            bufs[i % _NBUF, base : base + 8, 0:128] += jnp.where(
                mask, 2.0 * key_ref[0], 0.0
            )
        start_out(i)
    for i in range(max(0, _N_CHUNKS - _NBUF), _N_CHUNKS):
        wait_out(i)


def kernel(key, cache_next):
    flat = cache_next.reshape(_FLAT_ROWS, _SHAPE[2])
    out = pl.pallas_call(
        _copy_update_kernel,
        out_shape=jax.ShapeDtypeStruct((_FLAT_ROWS, _SHAPE[2]), jnp.float32),
        in_specs=[
            pl.BlockSpec(memory_space=pltpu.SMEM),
            pl.BlockSpec(memory_space=pl.ANY),
        ],
        out_specs=pl.BlockSpec(memory_space=pl.ANY),
        scratch_shapes=[
            pltpu.VMEM((_NBUF, _MAX_ROWS, _SHAPE[2]), jnp.float32),
            pltpu.SemaphoreType.DMA((_NBUF,)),
            pltpu.SemaphoreType.DMA((_NBUF,)),
        ],
    )(key, flat)
    return key, out.reshape(_SHAPE)
